# baseline (speedup 1.0000x reference)
import jax
import jax.numpy as jnp
from jax import lax
from jax.experimental import pallas as pl
from jax.experimental.pallas import tpu as pltpu
from jax.experimental.pallas import tpu_sc as plsc

_DIM = 512
_VOCAB = 209
_P = 16
_NW = 32
_NC = 2
_CH = 16
_LANES = 16


def _body(idx_hbm, emb_hbm, *rest):
    outs = rest[:_P]
    tab_v, idx_v, buf0, buf1, sem0, sem1 = rest[_P:]
    bufs = (buf0, buf1)
    sems = (sem0, sem1)

    n_idx = idx_v.shape[0]
    rows_per_w = n_idx // _P
    nch_s = rows_per_w // _CH
    tot = n_idx // _CH
    chunk_elems = _CH * _DIM

    wid = lax.axis_index("s") * _NC + lax.axis_index("c")
    base = wid * rows_per_w

    pltpu.sync_copy(emb_hbm, tab_v)
    pltpu.sync_copy(idx_hbm.at[wid], idx_v)

    lane = jnp.arange(_LANES, dtype=jnp.int32)

    def construct(buf, k):
        idxvec = idx_v[pl.ds(k * _CH, _CH)]
        for i in range(_CH):
            bcast = idxvec.at[jnp.full((_LANES,), i, jnp.int32)].get(
                mode="promise_in_bounds")
            rowoff = bcast * _DIM + lane
            for j in range(_DIM // _LANES):
                buf[pl.ds(i * _DIM + j * _LANES, _LANES)] = (
                    plsc.load_gather(tab_v, [rowoff + j * _LANES]))

    def fire_scatter(b, k):
        s = k // nch_s
        dst = (base + (k % nch_s) * _CH) * _DIM
        for si in range(_P):
            @pl.when(s == si)
            def _():
                pltpu.async_copy(
                    bufs[b], outs[si].at[pl.ds(dst, chunk_elems)], sems[b])

    def wait_buf(b):
        pltpu.make_async_copy(
            bufs[b], outs[0].at[pl.ds(0, chunk_elems)], sems[b]).wait()

    for b in range(2):
        pltpu.async_copy(
            bufs[b], outs[0].at[pl.ds((base + b * _CH) * _DIM, chunk_elems)],
            sems[b])

    @pl.loop(0, tot, step=2)
    def _(k0):
        for b in range(2):
            k = k0 + b
            wait_buf(b)
            construct(bufs[b], k)
            fire_scatter(b, k)

    for b in range(2):
        wait_buf(b)


def kernel(path, embed):
    b, s, p = path.shape
    n = b * s
    rows_per_w = n // _NW

    idx = jnp.transpose(path.reshape(n, p)).reshape(p, _NW, rows_per_w)
    idx = jnp.transpose(idx, (1, 0, 2)).reshape(_NW, p * rows_per_w)

    mesh = plsc.VectorSubcoreMesh(core_axis_name="c", subcore_axis_name="s")
    run = pl.kernel(
        _body,
        out_type=[jax.ShapeDtypeStruct((n * _DIM,), jnp.float32)] * _P,
        mesh=mesh,
        compiler_params=pltpu.CompilerParams(needs_layout_passes=False),
        scratch_types=(
            [pltpu.VMEM((_VOCAB * _DIM,), jnp.float32),
             pltpu.VMEM((p * rows_per_w,), jnp.int32),
             pltpu.VMEM((_CH * _DIM,), jnp.float32),
             pltpu.VMEM((_CH * _DIM,), jnp.float32)]
            + [pltpu.SemaphoreType.DMA] * 2
        ),
    )
    outs = run(idx, embed.reshape(-1))
    return tuple(o.reshape(b, s, _DIM) for o in outs)


# speedup vs baseline: 2.3778x; 2.3778x over previous
import jax
import jax.numpy as jnp
from jax import lax
from jax.experimental import pallas as pl
from jax.experimental.pallas import tpu as pltpu
from jax.experimental.pallas import tpu_sc as plsc

_DIM = 512
_VOCAB = 209
_P = 16
_NW = 32
_NC = 2
_CH = 16
_LANES = 16


def _body(idx_hbm, emb_hbm, *rest):
    outs = rest[:_P]
    tab_v, idx_v, buf0, buf1, sem0, sem1 = rest[_P:]
    bufs = (buf0, buf1)
    sems = (sem0, sem1)

    n_idx = idx_v.shape[0]
    rows_per_w = n_idx // _P
    nch_s = rows_per_w // _CH
    tot = n_idx // _CH
    chunk_elems = _CH * _DIM

    wid = lax.axis_index("s") * _NC + lax.axis_index("c")
    base = wid * rows_per_w

    pltpu.sync_copy(emb_hbm, tab_v)
    pltpu.sync_copy(idx_hbm.at[wid], idx_v)

    lane = jnp.arange(_LANES, dtype=jnp.int32)

    def construct(buf, k):
        idxvec = idx_v[pl.ds(k * _CH, _CH)]
        for i in range(_CH):
            bcast = idxvec.at[jnp.full((_LANES,), i, jnp.int32)].get(
                mode="promise_in_bounds")
            rowoff = bcast * _DIM + lane

            @plsc.parallel_loop(0, _DIM // _LANES, step=1, unroll=8)
            def _(j):
                buf[pl.ds(i * _DIM + j * _LANES, _LANES)] = (
                    plsc.load_gather(tab_v, [rowoff + j * _LANES]))

    def fire_scatter(b, k):
        s = k // nch_s
        dst = (base + (k % nch_s) * _CH) * _DIM
        for si in range(_P):
            @pl.when(s == si)
            def _():
                pltpu.async_copy(
                    bufs[b], outs[si].at[pl.ds(dst, chunk_elems)], sems[b])

    def wait_buf(b):
        pltpu.make_async_copy(
            bufs[b], outs[0].at[pl.ds(0, chunk_elems)], sems[b]).wait()

    for b in range(2):
        pltpu.async_copy(
            bufs[b], outs[0].at[pl.ds((base + b * _CH) * _DIM, chunk_elems)],
            sems[b])

    @pl.loop(0, tot, step=2)
    def _(k0):
        for b in range(2):
            k = k0 + b
            wait_buf(b)
            construct(bufs[b], k)
            fire_scatter(b, k)

    for b in range(2):
        wait_buf(b)


def kernel(path, embed):
    b, s, p = path.shape
    n = b * s
    rows_per_w = n // _NW

    idx = jnp.transpose(path.reshape(n, p)).reshape(p, _NW, rows_per_w)
    idx = jnp.transpose(idx, (1, 0, 2)).reshape(_NW, p * rows_per_w)

    mesh = plsc.VectorSubcoreMesh(core_axis_name="c", subcore_axis_name="s")
    run = pl.kernel(
        _body,
        out_type=[jax.ShapeDtypeStruct((n * _DIM,), jnp.float32)] * _P,
        mesh=mesh,
        compiler_params=pltpu.CompilerParams(needs_layout_passes=False),
        scratch_types=(
            [pltpu.VMEM((_VOCAB * _DIM,), jnp.float32),
             pltpu.VMEM((p * rows_per_w,), jnp.int32),
             pltpu.VMEM((_CH * _DIM,), jnp.float32),
             pltpu.VMEM((_CH * _DIM,), jnp.float32)]
            + [pltpu.SemaphoreType.DMA] * 2
        ),
    )
    outs = run(idx, embed.reshape(-1))
    return tuple(o.reshape(b, s, _DIM) for o in outs)


# speedup vs baseline: 2.4453x; 1.0284x over previous
import jax
import jax.numpy as jnp
from jax import lax
from jax.experimental import pallas as pl
from jax.experimental.pallas import tpu as pltpu
from jax.experimental.pallas import tpu_sc as plsc

_DIM = 512
_VOCAB = 209
_P = 16
_NW = 32
_NC = 2
_CH = 16
_LANES = 16


def _body(idx_hbm, emb_hbm, *rest):
    outs = rest[:_P]
    tab_v, idx_v, buf0, buf1, sem0, sem1 = rest[_P:]
    bufs = (buf0, buf1)
    sems = (sem0, sem1)

    n_idx = idx_v.shape[0]
    rows_per_w = n_idx // _P
    nch_s = rows_per_w // _CH
    tot = n_idx // _CH
    chunk_elems = _CH * _DIM

    wid = lax.axis_index("s") * _NC + lax.axis_index("c")
    base = wid * rows_per_w

    pltpu.sync_copy(emb_hbm, tab_v)
    pltpu.sync_copy(idx_hbm.at[wid], idx_v)

    lane = jnp.arange(_LANES, dtype=jnp.int32)

    def construct(buf, k):
        idxvec = idx_v[pl.ds(k * _CH, _CH)]

        @plsc.parallel_loop(0, _CH, step=1, unroll=2)
        def _(i):
            bcast = idxvec.at[jnp.full((_LANES,), i, jnp.int32)].get(
                mode="promise_in_bounds")
            rowoff = bcast * _DIM + lane
            for j in range(_DIM // _LANES):
                buf[pl.ds(i * _DIM + j * _LANES, _LANES)] = (
                    plsc.load_gather(tab_v, [rowoff + j * _LANES]))

    def fire_scatter(b, k):
        s = k // nch_s
        dst = (base + (k % nch_s) * _CH) * _DIM
        for si in range(_P):
            @pl.when(s == si)
            def _():
                pltpu.async_copy(
                    bufs[b], outs[si].at[pl.ds(dst, chunk_elems)], sems[b])

    def wait_buf(b):
        pltpu.make_async_copy(
            bufs[b], outs[0].at[pl.ds(0, chunk_elems)], sems[b]).wait()

    for b in range(2):
        pltpu.async_copy(
            bufs[b], outs[0].at[pl.ds((base + b * _CH) * _DIM, chunk_elems)],
            sems[b])

    @pl.loop(0, tot, step=2)
    def _(k0):
        for b in range(2):
            k = k0 + b
            wait_buf(b)
            construct(bufs[b], k)
            fire_scatter(b, k)

    for b in range(2):
        wait_buf(b)


def kernel(path, embed):
    b, s, p = path.shape
    n = b * s
    rows_per_w = n // _NW

    idx = jnp.transpose(path.reshape(n, p)).reshape(p, _NW, rows_per_w)
    idx = jnp.transpose(idx, (1, 0, 2)).reshape(_NW, p * rows_per_w)

    mesh = plsc.VectorSubcoreMesh(core_axis_name="c", subcore_axis_name="s")
    run = pl.kernel(
        _body,
        out_type=[jax.ShapeDtypeStruct((n * _DIM,), jnp.float32)] * _P,
        mesh=mesh,
        compiler_params=pltpu.CompilerParams(needs_layout_passes=False),
        scratch_types=(
            [pltpu.VMEM((_VOCAB * _DIM,), jnp.float32),
             pltpu.VMEM((p * rows_per_w,), jnp.int32),
             pltpu.VMEM((_CH * _DIM,), jnp.float32),
             pltpu.VMEM((_CH * _DIM,), jnp.float32)]
            + [pltpu.SemaphoreType.DMA] * 2
        ),
    )
    outs = run(idx, embed.reshape(-1))
    return tuple(o.reshape(b, s, _DIM) for o in outs)


# speedup vs baseline: 3.6566x; 1.4953x over previous
import jax
import jax.numpy as jnp
from jax import lax
from jax.experimental import pallas as pl
from jax.experimental.pallas import tpu as pltpu
from jax.experimental.pallas import tpu_sc as plsc

_DIM = 512
_VOCAB = 209
_P = 16
_NW = 32
_NC = 2
_CH = 16
_LANES = 16


def _body(idx_hbm, emb2d_hbm, emb_flat_hbm, *rest):
    outs = rest[:_P]
    tab_v, idx_v, buf_s, buf_c, sem_g, sem_ss, sem_sc = rest[_P:]

    n_idx = idx_v.shape[0]
    rows_per_w = n_idx // _P
    n_g = rows_per_w // (2 * _CH)

    wid = lax.axis_index("s") * _NC + lax.axis_index("c")
    base = wid * rows_per_w

    pltpu.sync_copy(emb_flat_hbm, tab_v)
    pltpu.sync_copy(idx_hbm.at[wid], idx_v)

    lane = jnp.arange(_LANES, dtype=jnp.int32)

    def construct(local):
        idxvec = idx_v[pl.ds(local, _CH)]

        @plsc.parallel_loop(0, _CH, step=1, unroll=2)
        def _(i):
            bcast = idxvec.at[jnp.full((_LANES,), i, jnp.int32)].get(
                mode="promise_in_bounds")
            rowoff = bcast * _DIM + lane
            for j in range(_DIM // _LANES):
                buf_c[i, pl.ds(j * _LANES, _LANES)] = (
                    plsc.load_gather(tab_v, [rowoff + j * _LANES]))

    def wait_sem(buf, sem):
        pltpu.make_async_copy(buf, outs[0].at[pl.ds(0, _CH)], sem).wait()

    pltpu.async_copy(buf_s, outs[0].at[pl.ds(base, _CH)], sem_ss)
    pltpu.async_copy(buf_c, outs[0].at[pl.ds(base + _CH, _CH)], sem_sc)

    @pl.loop(0, _P * n_g, unroll=2)
    def _(gk):
        slot = gk // n_g
        local = gk * (2 * _CH)
        row = base + (gk % n_g) * (2 * _CH)
        wait_sem(buf_s, sem_ss)
        gh = pltpu.async_copy(
            emb2d_hbm.at[idx_v.at[pl.ds(local, _CH)]], buf_s, sem_g)
        wait_sem(buf_c, sem_sc)
        construct(local + _CH)
        for si in range(_P):
            @pl.when(slot == si)
            def _():
                pltpu.async_copy(
                    buf_c, outs[si].at[pl.ds(row + _CH, _CH)], sem_sc)
        gh.wait()
        for si in range(_P):
            @pl.when(slot == si)
            def _():
                pltpu.async_copy(
                    buf_s, outs[si].at[pl.ds(row, _CH)], sem_ss)

    wait_sem(buf_s, sem_ss)
    wait_sem(buf_c, sem_sc)


def kernel(path, embed):
    b, s, p = path.shape
    n = b * s
    rows_per_w = n // _NW

    idx = jnp.transpose(path.reshape(n, p)).reshape(p, _NW, rows_per_w)
    idx = jnp.transpose(idx, (1, 0, 2)).reshape(_NW, p * rows_per_w)

    mesh = plsc.VectorSubcoreMesh(core_axis_name="c", subcore_axis_name="s")
    run = pl.kernel(
        _body,
        out_type=[jax.ShapeDtypeStruct((n, _DIM), jnp.float32)] * _P,
        mesh=mesh,
        compiler_params=pltpu.CompilerParams(needs_layout_passes=False),
        scratch_types=(
            [pltpu.VMEM((_VOCAB * _DIM,), jnp.float32),
             pltpu.VMEM((p * rows_per_w,), jnp.int32),
             pltpu.VMEM((_CH, _DIM), jnp.float32),
             pltpu.VMEM((_CH, _DIM), jnp.float32)]
            + [pltpu.SemaphoreType.DMA] * 3
        ),
    )
    outs = run(idx, embed, embed.reshape(-1))
    return tuple(o.reshape(b, s, _DIM) for o in outs)


# speedup vs baseline: 3.8049x; 1.0406x over previous
import jax
import jax.numpy as jnp
from jax import lax
from jax.experimental import pallas as pl
from jax.experimental.pallas import tpu as pltpu
from jax.experimental.pallas import tpu_sc as plsc

_DIM = 512
_VOCAB = 209
_P = 16
_NW = 32
_NC = 2
_CH = 16
_LANES = 16


def _body(idx_hbm, emb2d_hbm, emb_flat_hbm, *rest):
    outs = rest[:_P]
    tab_v, idx_v, buf_s, buf_c, sem_g, sem_ss, sem_sc = rest[_P:]

    n_idx = idx_v.shape[0]
    rows_per_w = n_idx // _P
    n_g = rows_per_w // (2 * _CH)

    wid = lax.axis_index("s") * _NC + lax.axis_index("c")
    base = wid * rows_per_w

    pltpu.sync_copy(emb_flat_hbm, tab_v)
    pltpu.sync_copy(idx_hbm.at[wid], idx_v)

    lane = jnp.arange(_LANES, dtype=jnp.int32)

    def construct(local):
        idxvec = idx_v[pl.ds(local, _CH)]

        @plsc.parallel_loop(0, _CH, step=1, unroll=2)
        def _(i):
            bcast = idxvec.at[jnp.full((_LANES,), i, jnp.int32)].get(
                mode="promise_in_bounds")
            rowoff = bcast * _DIM + lane
            for j in range(_DIM // _LANES):
                buf_c[i, pl.ds(j * _LANES, _LANES)] = (
                    plsc.load_gather(tab_v, [rowoff + j * _LANES]))

    def wait_sem(buf, sem):
        pltpu.make_async_copy(buf, outs[0].at[pl.ds(0, _CH)], sem).wait()

    pltpu.async_copy(buf_s, outs[0].at[pl.ds(base, _CH)], sem_ss)
    pltpu.async_copy(buf_c, outs[0].at[pl.ds(base + _CH, _CH)], sem_sc)

    @pl.loop(0, _P * n_g)
    def _(gk):
        slot = gk // n_g
        local = gk * (2 * _CH)
        row = base + (gk % n_g) * (2 * _CH)
        wait_sem(buf_s, sem_ss)
        gh = pltpu.async_copy(
            emb2d_hbm.at[idx_v.at[pl.ds(local, _CH)]], buf_s, sem_g)
        wait_sem(buf_c, sem_sc)
        construct(local + _CH)
        for si in range(_P):
            @pl.when(slot == si)
            def _():
                pltpu.async_copy(
                    buf_c, outs[si].at[pl.ds(row + _CH, _CH)], sem_sc)
        gh.wait()
        for si in range(_P):
            @pl.when(slot == si)
            def _():
                pltpu.async_copy(
                    buf_s, outs[si].at[pl.ds(row, _CH)], sem_ss)

    wait_sem(buf_s, sem_ss)
    wait_sem(buf_c, sem_sc)


def kernel(path, embed):
    b, s, p = path.shape
    n = b * s
    rows_per_w = n // _NW

    idx = jnp.transpose(path.reshape(n, p)).reshape(p, _NW, rows_per_w)
    idx = jnp.transpose(idx, (1, 0, 2)).reshape(_NW, p * rows_per_w)

    mesh = plsc.VectorSubcoreMesh(core_axis_name="c", subcore_axis_name="s")
    run = pl.kernel(
        _body,
        out_type=[jax.ShapeDtypeStruct((n, _DIM), jnp.float32)] * _P,
        mesh=mesh,
        compiler_params=pltpu.CompilerParams(needs_layout_passes=False),
        scratch_types=(
            [pltpu.VMEM((_VOCAB * _DIM,), jnp.float32),
             pltpu.VMEM((p * rows_per_w,), jnp.int32),
             pltpu.VMEM((_CH, _DIM), jnp.float32),
             pltpu.VMEM((_CH, _DIM), jnp.float32)]
            + [pltpu.SemaphoreType.DMA] * 3
        ),
    )
    outs = run(idx, embed, embed.reshape(-1))
    return tuple(o.reshape(b, s, _DIM) for o in outs)
